# Initial kernel scaffold; baseline (speedup 1.0000x reference)
import jax
import jax.numpy as jnp
from jax.experimental import pallas as pl


def kernel(x, edge_index, batch, W1, b1, W2, b2, Wfc, bfc):
    raise NotImplementedError("write your pallas kernel here")


# speedup vs baseline: 28.8546x; 28.8546x over previous
import functools

import jax
import jax.numpy as jnp
from jax import lax
from jax.experimental import pallas as pl
from jax.experimental.pallas import tpu as pltpu
from jax.experimental.pallas import tpu_sc as plsc

N = 10000
E = 320000
D = 128
H = 16
NG = 64
NCLS = 2

NC, NS, L = 2, 16, 16
NW = NC * NS
EROW = 128
RPT = 80
EPAD = NW * RPT * EROW
NPAD = 10112
STRIPE = NPAD // NS
TRASH = 10008
ZROW = 10000

def _deg_body(dst_hbm, ones_hbm, zeros_hbm, out_hbm, idx_v, ones_v, acc_sh):
    c = lax.axis_index("c")
    s = lax.axis_index("s")
    wid = s * NC + c
    pltpu.sync_copy(zeros_hbm, acc_sh.at[pl.ds(s * STRIPE, STRIPE)])
    pltpu.sync_copy(ones_hbm, ones_v)
    pltpu.sync_copy(dst_hbm.at[pl.ds(wid * RPT, RPT)], idx_v)
    plsc.subcore_barrier()

    def body(j, carry):
        pltpu.sync_copy(ones_v, acc_sh.at[idx_v.at[j]], add=True)
        return carry

    lax.fori_loop(0, RPT, body, 0)
    plsc.subcore_barrier()
    pltpu.sync_copy(
        acc_sh.at[pl.ds(s * STRIPE, STRIPE)],
        out_hbm.at[c, pl.ds(s * STRIPE, STRIPE)],
    )


def _mp_body(g_hbm, src_hbm, dst_hbm, zeros_hbm, out_hbm,
             src_v, dst_v, rows_v, acc_sh):
    c = lax.axis_index("c")
    s = lax.axis_index("s")
    wid = s * NC + c
    pltpu.sync_copy(zeros_hbm, acc_sh.at[pl.ds(s * STRIPE, STRIPE)])
    pltpu.sync_copy(src_hbm.at[pl.ds(wid * RPT, RPT)], src_v)
    pltpu.sync_copy(dst_hbm.at[pl.ds(wid * RPT, RPT)], dst_v)
    plsc.subcore_barrier()

    def body(j, carry):
        pltpu.sync_copy(g_hbm.at[src_v.at[j]], rows_v)
        pltpu.sync_copy(rows_v, acc_sh.at[dst_v.at[j]], add=True)
        return carry

    lax.fori_loop(0, RPT, body, 0)
    plsc.subcore_barrier()
    pltpu.sync_copy(
        acc_sh.at[pl.ds(s * STRIPE, STRIPE)],
        out_hbm.at[c, pl.ds(s * STRIPE, STRIPE)],
    )


def _t1_body(x_ref, w1_ref, dego_ref, g1_ref, dinv_ref):
    deg = dego_ref[0] + dego_ref[1] + 1.0
    dinv = lax.rsqrt(deg)
    h = jnp.dot(x_ref[...], w1_ref[...], preferred_element_type=jnp.float32)
    rid = lax.broadcasted_iota(jnp.int32, (NPAD, H), 0)
    mask = rid < N
    g1_ref[...] = jnp.where(mask, h * dinv, 0.0)
    dinv_ref[...] = jnp.where(mask, dinv, 0.0)


def _t2_body(agg_ref, g1_ref, dinv_ref, w2_ref, b1_ref, g2_ref):
    aggs = agg_ref[0] + agg_ref[1]
    dinv = dinv_ref[...]
    z = jnp.maximum(dinv * (aggs + g1_ref[...]) + b1_ref[...], 0.0)
    h2 = jnp.dot(z, w2_ref[...], preferred_element_type=jnp.float32)
    rid = lax.broadcasted_iota(jnp.int32, (NPAD, H), 0)
    g2_ref[...] = jnp.where(rid < N, h2 * dinv, 0.0)


def _t3_body(agg_ref, g2_ref, dinv_ref, b2_ref, batch_ref, wfc_ref, bfc_ref,
             out_ref):
    aggs = agg_ref[0] + agg_ref[1]
    z = jnp.maximum(dinv_ref[...] * (aggs + g2_ref[...]) + b2_ref[...], 0.0)
    seg = lax.broadcasted_iota(jnp.int32, (NG, NPAD), 0)
    onehot = jnp.where(seg == batch_ref[...], 1.0, 0.0)
    sums = jnp.dot(onehot, z, preferred_element_type=jnp.float32)
    cnts = jnp.sum(onehot, axis=1, keepdims=True)
    pooled = sums / jnp.maximum(cnts, 1.0)
    out_ref[...] = (
        jnp.dot(pooled, wfc_ref[...], preferred_element_type=jnp.float32)
        + bfc_ref[...]
    )


@functools.lru_cache(maxsize=None)
def _sc_kernels():
    mesh = plsc.VectorSubcoreMesh(core_axis_name="c", subcore_axis_name="s")
    params = pltpu.CompilerParams(use_tc_tiling_on_sc=False)
    deg = pl.kernel(
        _deg_body,
        out_type=jax.ShapeDtypeStruct((NC, NPAD, H), jnp.float32),
        mesh=mesh,
        scratch_types=[
            pltpu.VMEM((RPT, EROW), jnp.int32),
            pltpu.VMEM((EROW, H), jnp.float32),
            pltpu.VMEM_SHARED((NPAD, H), jnp.float32),
        ],
        compiler_params=params,
    )
    mp = pl.kernel(
        _mp_body,
        out_type=jax.ShapeDtypeStruct((NC, NPAD, H), jnp.float32),
        mesh=mesh,
        scratch_types=[
            pltpu.VMEM((RPT, EROW), jnp.int32),
            pltpu.VMEM((RPT, EROW), jnp.int32),
            pltpu.VMEM((EROW, H), jnp.float32),
            pltpu.VMEM_SHARED((NPAD, H), jnp.float32),
        ],
        compiler_params=params,
    )
    return deg, mp


_t1_call = pl.pallas_call(
    _t1_body,
    out_shape=(
        jax.ShapeDtypeStruct((NPAD, H), jnp.float32),
        jax.ShapeDtypeStruct((NPAD, H), jnp.float32),
    ),
)

_t2_call = pl.pallas_call(
    _t2_body,
    out_shape=jax.ShapeDtypeStruct((NPAD, H), jnp.float32),
)

_t3_call = pl.pallas_call(
    _t3_body,
    out_shape=jax.ShapeDtypeStruct((NG, NCLS), jnp.float32),
)


@jax.jit
def kernel(x, edge_index, batch, W1, b1, W2, b2, Wfc, bfc):
    src = edge_index[0]
    dst = edge_index[1]
    pad_e = EPAD - E
    srcp = jnp.concatenate(
        [src, jnp.full((pad_e,), ZROW, jnp.int32)]).reshape(NW * RPT, EROW)
    dstp = jnp.concatenate(
        [dst, jnp.full((pad_e,), TRASH, jnp.int32)]).reshape(NW * RPT, EROW)
    zeros_stripe = jnp.zeros((STRIPE, H), jnp.float32)
    ones_rows = jnp.ones((EROW, H), jnp.float32)
    xp = jnp.pad(x, ((0, NPAD - N), (0, 0)))
    batchp = jnp.concatenate(
        [batch, jnp.full((NPAD - N,), NG, jnp.int32)]).reshape(1, NPAD)

    deg_kernel, mp_kernel = _sc_kernels()
    dego = deg_kernel(dstp, ones_rows, zeros_stripe)
    g1, dinv = _t1_call(xp, W1, dego)
    agg1 = mp_kernel(g1, srcp, dstp, zeros_stripe)
    g2 = _t2_call(agg1, g1, dinv, W2, b1.reshape(1, H))
    agg2 = mp_kernel(g2, srcp, dstp, zeros_stripe)
    return _t3_call(agg2, g2, dinv, b2.reshape(1, H), batchp, Wfc,
                    bfc.reshape(1, NCLS))


# speedup vs baseline: 35.8552x; 1.2426x over previous
import functools

import jax
import jax.numpy as jnp
from jax import lax
from jax.experimental import pallas as pl
from jax.experimental.pallas import tpu as pltpu
from jax.experimental.pallas import tpu_sc as plsc

N = 10000
E = 320000
D = 128
H = 16
NG = 64
NCLS = 2

NC, NS, L = 2, 16, 16
NW = NC * NS
EROW = 128
RPT = 80
EPAD = NW * RPT * EROW
NPAD = 10112
STRIPE = NPAD // NS
TRASH = 10008
ZROW = 10000

DEG_K = 8


def _deg_body(dst_hbm, ones_hbm, zeros_hbm, out_hbm, idx_v, ones_v, acc_sh,
              ssem):
    c = lax.axis_index("c")
    s = lax.axis_index("s")
    wid = s * NC + c
    pltpu.sync_copy(zeros_hbm, acc_sh.at[pl.ds(s * STRIPE, STRIPE)])
    pltpu.sync_copy(ones_hbm, ones_v)
    pltpu.sync_copy(dst_hbm.at[pl.ds(wid * RPT, RPT)], idx_v)
    plsc.subcore_barrier()

    def body(i, carry):
        descs = [
            pltpu.async_copy(ones_v, acc_sh.at[idx_v.at[i * DEG_K + j]],
                             ssem, add=True)
            for j in range(DEG_K)
        ]
        for d in descs:
            d.wait()
        return carry

    lax.fori_loop(0, RPT // DEG_K, body, 0)
    plsc.subcore_barrier()
    pltpu.sync_copy(
        acc_sh.at[pl.ds(s * STRIPE, STRIPE)],
        out_hbm.at[c, pl.ds(s * STRIPE, STRIPE)],
    )


MP_K = 4
MP_NB2 = RPT // (2 * MP_K)


def _mp_body(g_hbm, src_hbm, dst_hbm, zeros_hbm, out_hbm,
             src_v, dst_v, buf_a, buf_b, acc_sh, gs_a, gs_b, ss_a, ss_b):
    c = lax.axis_index("c")
    s = lax.axis_index("s")
    wid = s * NC + c
    pltpu.sync_copy(zeros_hbm, acc_sh.at[pl.ds(s * STRIPE, STRIPE)])
    pltpu.sync_copy(src_hbm.at[pl.ds(wid * RPT, RPT)], src_v)
    pltpu.sync_copy(dst_hbm.at[pl.ds(wid * RPT, RPT)], dst_v)
    plsc.subcore_barrier()

    def fire_g(blk, buf, sem):
        return [
            pltpu.async_copy(g_hbm.at[src_v.at[blk * MP_K + j]],
                             buf.at[pl.ds(j * EROW, EROW)], sem)
            for j in range(MP_K)
        ]

    def fire_s(blk, buf, sem):
        return [
            pltpu.async_copy(buf.at[pl.ds(j * EROW, EROW)],
                             acc_sh.at[dst_v.at[blk * MP_K + j]], sem,
                             add=True)
            for j in range(MP_K)
        ]

    def body(i, carry):
        b0 = 2 * i
        b1 = b0 + 1
        g_a = fire_g(b0, buf_a, gs_a)
        g_b = fire_g(b1, buf_b, gs_b)
        for d in g_a:
            d.wait()
        s_a = fire_s(b0, buf_a, ss_a)
        for d in g_b:
            d.wait()
        s_b = fire_s(b1, buf_b, ss_b)
        for d in s_a:
            d.wait()
        for d in s_b:
            d.wait()
        return carry

    lax.fori_loop(0, MP_NB2, body, 0)
    plsc.subcore_barrier()
    pltpu.sync_copy(
        acc_sh.at[pl.ds(s * STRIPE, STRIPE)],
        out_hbm.at[c, pl.ds(s * STRIPE, STRIPE)],
    )


def _t1_body(x_ref, w1_ref, dego_ref, g1_ref, dinv_ref):
    deg = dego_ref[0] + dego_ref[1] + 1.0
    dinv = lax.rsqrt(deg)
    h = jnp.dot(x_ref[...], w1_ref[...], preferred_element_type=jnp.float32)
    rid = lax.broadcasted_iota(jnp.int32, (NPAD, H), 0)
    mask = rid < N
    g1_ref[...] = jnp.where(mask, h * dinv, 0.0)
    dinv_ref[...] = jnp.where(mask, dinv, 0.0)


def _t2_body(agg_ref, g1_ref, dinv_ref, w2_ref, b1_ref, g2_ref):
    aggs = agg_ref[0] + agg_ref[1]
    dinv = dinv_ref[...]
    z = jnp.maximum(dinv * (aggs + g1_ref[...]) + b1_ref[...], 0.0)
    h2 = jnp.dot(z, w2_ref[...], preferred_element_type=jnp.float32)
    rid = lax.broadcasted_iota(jnp.int32, (NPAD, H), 0)
    g2_ref[...] = jnp.where(rid < N, h2 * dinv, 0.0)


def _t3_body(agg_ref, g2_ref, dinv_ref, b2_ref, batch_ref, wfc_ref, bfc_ref,
             out_ref):
    aggs = agg_ref[0] + agg_ref[1]
    z = jnp.maximum(dinv_ref[...] * (aggs + g2_ref[...]) + b2_ref[...], 0.0)
    seg = lax.broadcasted_iota(jnp.int32, (NG, NPAD), 0)
    onehot = jnp.where(seg == batch_ref[...], 1.0, 0.0)
    sums = jnp.dot(onehot, z, preferred_element_type=jnp.float32)
    cnts = jnp.sum(onehot, axis=1, keepdims=True)
    pooled = sums / jnp.maximum(cnts, 1.0)
    out_ref[...] = (
        jnp.dot(pooled, wfc_ref[...], preferred_element_type=jnp.float32)
        + bfc_ref[...]
    )


@functools.lru_cache(maxsize=None)
def _sc_kernels():
    mesh = plsc.VectorSubcoreMesh(core_axis_name="c", subcore_axis_name="s")
    params = pltpu.CompilerParams(use_tc_tiling_on_sc=False)
    deg = pl.kernel(
        _deg_body,
        out_type=jax.ShapeDtypeStruct((NC, NPAD, H), jnp.float32),
        mesh=mesh,
        scratch_types=[
            pltpu.VMEM((RPT, EROW), jnp.int32),
            pltpu.VMEM((EROW, H), jnp.float32),
            pltpu.VMEM_SHARED((NPAD, H), jnp.float32),
            pltpu.SemaphoreType.DMA,
        ],
        compiler_params=params,
    )
    mp = pl.kernel(
        _mp_body,
        out_type=jax.ShapeDtypeStruct((NC, NPAD, H), jnp.float32),
        mesh=mesh,
        scratch_types=[
            pltpu.VMEM((RPT, EROW), jnp.int32),
            pltpu.VMEM((RPT, EROW), jnp.int32),
            pltpu.VMEM((MP_K * EROW, H), jnp.float32),
            pltpu.VMEM((MP_K * EROW, H), jnp.float32),
            pltpu.VMEM_SHARED((NPAD, H), jnp.float32),
            pltpu.SemaphoreType.DMA,
            pltpu.SemaphoreType.DMA,
            pltpu.SemaphoreType.DMA,
            pltpu.SemaphoreType.DMA,
        ],
        compiler_params=params,
    )
    return deg, mp


_t1_call = pl.pallas_call(
    _t1_body,
    out_shape=(
        jax.ShapeDtypeStruct((NPAD, H), jnp.float32),
        jax.ShapeDtypeStruct((NPAD, H), jnp.float32),
    ),
)

_t2_call = pl.pallas_call(
    _t2_body,
    out_shape=jax.ShapeDtypeStruct((NPAD, H), jnp.float32),
)

_t3_call = pl.pallas_call(
    _t3_body,
    out_shape=jax.ShapeDtypeStruct((NG, NCLS), jnp.float32),
)


@jax.jit
def kernel(x, edge_index, batch, W1, b1, W2, b2, Wfc, bfc):
    src = edge_index[0]
    dst = edge_index[1]
    pad_e = EPAD - E
    srcp = jnp.concatenate(
        [src, jnp.full((pad_e,), ZROW, jnp.int32)]).reshape(NW * RPT, EROW)
    dstp = jnp.concatenate(
        [dst, jnp.full((pad_e,), TRASH, jnp.int32)]).reshape(NW * RPT, EROW)
    zeros_stripe = jnp.zeros((STRIPE, H), jnp.float32)
    ones_rows = jnp.ones((EROW, H), jnp.float32)
    xp = jnp.pad(x, ((0, NPAD - N), (0, 0)))
    batchp = jnp.concatenate(
        [batch, jnp.full((NPAD - N,), NG, jnp.int32)]).reshape(1, NPAD)

    deg_kernel, mp_kernel = _sc_kernels()
    dego = deg_kernel(dstp, ones_rows, zeros_stripe)
    g1, dinv = _t1_call(xp, W1, dego)
    agg1 = mp_kernel(g1, srcp, dstp, zeros_stripe)
    g2 = _t2_call(agg1, g1, dinv, W2, b1.reshape(1, H))
    agg2 = mp_kernel(g2, srcp, dstp, zeros_stripe)
    return _t3_call(agg2, g2, dinv, b2.reshape(1, H), batchp, Wfc,
                    bfc.reshape(1, NCLS))
